# baseline (speedup 1.0000x reference)
import functools

import jax
import jax.numpy as jnp
from jax import lax
from jax.experimental import pallas as pl
from jax.experimental.pallas import tpu as pltpu
from jax.experimental.pallas import tpu_sc as plsc

B = 16384
N_FIELDS = 26
VOCAB = 100000
EMB_DIM = 32

NC = 2
NS = 16
LANES = 16
NW = NC * NS

ROWS = N_FIELDS * EMB_DIM
PER_W = ROWS // NW
BH = 8192
UNROLL = 8

TCH = 33280
TAIL = VOCAB - 3 * TCH
TBUF = TCH + TAIL


@functools.partial(
    pl.kernel,
    out_type=jax.ShapeDtypeStruct((ROWS, B), jnp.float32),
    mesh=plsc.VectorSubcoreMesh(
        core_axis_name="c", subcore_axis_name="s", num_cores=NC, num_subcores=NS
    ),
    scratch_types=[
        pltpu.VMEM((1, TBUF), jnp.float32),
        pltpu.VMEM((1, TBUF), jnp.float32),
        pltpu.VMEM((B,), jnp.int32),
        pltpu.VMEM((BH,), jnp.float32),
        pltpu.VMEM((BH,), jnp.float32),
        pltpu.SemaphoreType.DMA,
        pltpu.SemaphoreType.DMA,
        pltpu.SemaphoreType.DMA,
        pltpu.SemaphoreType.DMA,
    ],
    compiler_params=pltpu.CompilerParams(
        use_tc_tiling_on_sc=True, needs_layout_passes=False
    ),
)
def _sc_col_gather(feat_hbm, table_hbm, tail_hbm, out_hbm, buf_a, buf_b,
                   fc_v, ob0, ob1, sem_a, sem_b, osem0, osem1):
    wid = lax.axis_index("s") * NC + lax.axis_index("c")
    p0 = wid * PER_W

    def plane_cp(p, third, buf, sem):
        return pltpu.make_async_copy(
            table_hbm.at[pl.ds(p, 1), pl.ds(third * TCH, TCH)],
            buf.at[pl.ds(0, 1), pl.ds(0, TCH)],
            sem,
        )

    def tail_cp(p, buf, sem):
        return pltpu.make_async_copy(
            tail_hbm.at[pl.ds(p, 1)],
            buf.at[pl.ds(0, 1), pl.ds(TCH, TAIL)],
            sem,
        )

    def out_cp(p, half, ob, sem):
        return pltpu.make_async_copy(
            ob, out_hbm.at[p, pl.ds(half * BH, BH)], sem
        )

    def merge_pass(buf, third, half, ob):
        lo = third * TCH
        hi = lo + (TCH + TAIL if third == 2 else TCH)
        zero = jnp.zeros((LANES,), jnp.int32)

        def body(g, carry):
            for u in range(UNROLL):
                off = half * BH + (g * UNROLL + u) * LANES
                idx = fc_v[pl.ds(off, LANES)]
                m = (idx >= lo) & (idx < hi)
                rel = jnp.where(m, idx - lo, 0)
                v = plsc.load_gather(buf, [zero, rel])
                cur = ob[pl.ds((g * UNROLL + u) * LANES, LANES)]
                ob[pl.ds((g * UNROLL + u) * LANES, LANES)] = jnp.where(m, v, cur)
            return carry

        lax.fori_loop(0, BH // (UNROLL * LANES), body, 0)

    def emit_pair(t, p, bx, sx, by, sy, first):
        if first:
            @pl.when(t > 0)
            def _():
                out_cp(p, 0, ob0, osem0).wait()
                out_cp(p, 1, ob1, osem1).wait()
        else:
            out_cp(p, 0, ob0, osem0).wait()
            out_cp(p, 1, ob1, osem1).wait()

        cond = (t == 0) | (p % EMB_DIM == 0) if first else (p % EMB_DIM == 0)

        @pl.when(cond)
        def _():
            pltpu.sync_copy(feat_hbm.at[p // EMB_DIM], fc_v)

        plane_cp(p, 0, bx, sx).wait()
        merge_pass(bx, 0, 0, ob0)
        merge_pass(bx, 0, 1, ob1)
        plane_cp(p, 2, bx, sx).start()
        tail_cp(p, bx, sx).start()

        plane_cp(p, 1, by, sy).wait()
        merge_pass(by, 1, 0, ob0)
        merge_pass(by, 1, 1, ob1)

        @pl.when(p + 1 < p0 + PER_W)
        def _():
            plane_cp(p + 1, 0, by, sy).start()

        plane_cp(p, 2, bx, sx).wait()
        tail_cp(p, bx, sx).wait()
        merge_pass(bx, 2, 0, ob0)
        merge_pass(bx, 2, 1, ob1)

        @pl.when(p + 1 < p0 + PER_W)
        def _():
            plane_cp(p + 1, 1, bx, sx).start()

        out_cp(p, 0, ob0, osem0).start()
        out_cp(p, 1, ob1, osem1).start()

    plane_cp(p0, 0, buf_a, sem_a).start()
    plane_cp(p0, 1, buf_b, sem_b).start()

    def step(t, carry):
        p = p0 + 2 * t
        emit_pair(t, p, buf_a, sem_a, buf_b, sem_b, first=True)
        emit_pair(t, p + 1, buf_b, sem_b, buf_a, sem_a, first=False)
        return carry

    lax.fori_loop(0, PER_W // 2, step, 0)
    out_cp(p0, 0, ob0, osem0).wait()
    out_cp(p0, 1, ob1, osem1).wait()


def kernel(features, tables):
    feat_t = features.T
    table_rows = tables.transpose(0, 2, 1).reshape(ROWS, VOCAB)
    tail_rows = lax.slice(table_rows, (0, 3 * TCH), (ROWS, VOCAB))
    out_t = _sc_col_gather(feat_t, table_rows, tail_rows)
    return out_t.T


# speedup vs baseline: 1.1038x; 1.1038x over previous
import functools

import jax
import jax.numpy as jnp
from jax import lax
from jax.experimental import pallas as pl
from jax.experimental.pallas import tpu as pltpu
from jax.experimental.pallas import tpu_sc as plsc

B = 16384
N_FIELDS = 26
VOCAB = 100000
EMB_DIM = 32

NC = 2
NS = 16
LANES = 16
NW = NC * NS

ROWS = N_FIELDS * EMB_DIM
PER_W = ROWS // NW
BH = 8192
UNROLL = 8

CH = 49920
TAIL = VOCAB - 2 * CH
TBUF = CH + TAIL


@functools.partial(
    pl.kernel,
    out_type=jax.ShapeDtypeStruct((ROWS, B), jnp.float32),
    mesh=plsc.VectorSubcoreMesh(
        core_axis_name="c", subcore_axis_name="s", num_cores=NC, num_subcores=NS
    ),
    scratch_types=[
        pltpu.VMEM((1, TBUF), jnp.float32),
        pltpu.VMEM((1, TBUF), jnp.float32),
        pltpu.VMEM((BH,), jnp.int32),
        pltpu.VMEM((BH,), jnp.float32),
        pltpu.VMEM((BH,), jnp.float32),
        pltpu.SemaphoreType.DMA,
        pltpu.SemaphoreType.DMA,
        pltpu.SemaphoreType.DMA,
        pltpu.SemaphoreType.DMA,
    ],
    compiler_params=pltpu.CompilerParams(
        use_tc_tiling_on_sc=True, needs_layout_passes=False
    ),
)
def _sc_col_gather(feat_hbm, table_hbm, tail_hbm, out_hbm, buf_a, buf_b,
                   fc_v, ob0, ob1, sem_a, sem_b, osem0, osem1):
    wid = lax.axis_index("s") * NC + lax.axis_index("c")
    p0 = wid * PER_W

    def chunk_cp(p, chunk, buf, sem):
        return pltpu.make_async_copy(
            table_hbm.at[pl.ds(p, 1), pl.ds(chunk * CH, CH)],
            buf.at[pl.ds(0, 1), pl.ds(0, CH)],
            sem,
        )

    def tail_cp(p, buf, sem):
        return pltpu.make_async_copy(
            tail_hbm.at[pl.ds(p, 1)],
            buf.at[pl.ds(0, 1), pl.ds(CH, TAIL)],
            sem,
        )

    def out_cp(half, ob, sem, p):
        return pltpu.make_async_copy(
            ob, out_hbm.at[p, pl.ds(half * BH, BH)], sem
        )

    def fc_load(p, half):
        pltpu.sync_copy(
            feat_hbm.at[p // EMB_DIM, pl.ds(half * BH, BH)], fc_v
        )

    def merge_pass(buf, rng, ob):
        lo = rng * CH
        hi = lo + (CH + TAIL if rng == 1 else CH)
        zero = jnp.zeros((LANES,), jnp.int32)

        def body(g, carry):
            for u in range(UNROLL):
                off = (g * UNROLL + u) * LANES
                idx = fc_v[pl.ds(off, LANES)]
                m = (idx >= lo) & (idx < hi)
                rel = jnp.where(m, idx - lo, 0)
                v = plsc.load_gather(buf, [zero, rel])
                cur = ob[pl.ds(off, LANES)]
                ob[pl.ds(off, LANES)] = jnp.where(m, v, cur)
            return carry

        lax.fori_loop(0, BH // (UNROLL * LANES), body, 0)

    chunk_cp(p0, 0, buf_a, sem_a).start()
    chunk_cp(p0, 1, buf_b, sem_b).start()
    tail_cp(p0, buf_b, sem_b).start()

    def step(k, carry):
        p = p0 + k

        @pl.when(k > 0)
        def _():
            out_cp(0, ob0, osem0, p).wait()
            out_cp(1, ob1, osem1, p).wait()

        chunk_cp(p, 0, buf_a, sem_a).wait()
        fc_load(p, 0)
        merge_pass(buf_a, 0, ob0)
        fc_load(p, 1)
        merge_pass(buf_a, 0, ob1)

        @pl.when(k + 1 < PER_W)
        def _():
            chunk_cp(p + 1, 0, buf_a, sem_a).start()

        chunk_cp(p, 1, buf_b, sem_b).wait()
        tail_cp(p, buf_b, sem_b).wait()
        merge_pass(buf_b, 1, ob1)
        fc_load(p, 0)
        merge_pass(buf_b, 1, ob0)

        @pl.when(k + 1 < PER_W)
        def _():
            chunk_cp(p + 1, 1, buf_b, sem_b).start()
            tail_cp(p + 1, buf_b, sem_b).start()

        out_cp(0, ob0, osem0, p).start()
        out_cp(1, ob1, osem1, p).start()
        return carry

    lax.fori_loop(0, PER_W, step, 0)
    out_cp(0, ob0, osem0, p0).wait()
    out_cp(1, ob1, osem1, p0).wait()


def kernel(features, tables):
    feat_t = features.T
    table_rows = tables.transpose(0, 2, 1).reshape(ROWS, VOCAB)
    tail_rows = lax.slice(table_rows, (0, 2 * CH), (ROWS, VOCAB))
    out_t = _sc_col_gather(feat_t, table_rows, tail_rows)
    return out_t.T
